# baseline (speedup 1.0000x reference)
import functools

import jax
import jax.numpy as jnp
from jax import lax
from jax.experimental import pallas as pl
from jax.experimental.pallas import tpu as pltpu
from jax.experimental.pallas import tpu_sc as plsc


def _build(B, S, D):
  info = plsc.get_sparse_core_info()
  NC, NS, L = info.num_cores, info.num_subcores, info.num_lanes
  NW = NC * NS
  assert B % (8 * NW) == 0, "batch must split 8-aligned across subcores"
  bpw = B // NW

  mesh = plsc.VectorSubcoreMesh(core_axis_name="c", subcore_axis_name="s")

  @functools.partial(
      pl.kernel,
      mesh=mesh,
      out_type=jax.ShapeDtypeStruct((B, 2 * D), jnp.float32),
      scratch_types=[
          pltpu.VMEM((bpw,), jnp.int32),
          pltpu.VMEM((bpw,), jnp.int32),
          pltpu.VMEM((bpw, D), jnp.float32),
          pltpu.VMEM((bpw, D), jnp.float32),
          pltpu.SemaphoreType.DMA,
          pltpu.SemaphoreType.DMA,
          pltpu.SemaphoreType.DMA,
      ],
  )
  def k(begin_hbm, end_hbm, fwd_hbm, bwd_hbm, out_hbm,
        bidx, eidx, brows, erows, isem, gsem, wsem):
    wid = lax.axis_index("c") * NS + lax.axis_index("s")
    base = wid * bpw
    ci = pltpu.async_copy(begin_hbm.at[pl.ds(base, bpw)], bidx, isem)
    cj = pltpu.async_copy(end_hbm.at[pl.ds(base, bpw)], eidx, isem)
    ci.wait()
    cj.wait()
    lane = lax.iota(jnp.int32, L)
    for j in range(bpw // L):
      sl = pl.ds(j * L, L)
      off = (base + j * L + lane) * S
      bidx[sl] = bidx[sl] + off
      eidx[sl] = eidx[sl] + off
    cb = pltpu.async_copy(bwd_hbm.at[bidx], brows, gsem)
    ce = pltpu.async_copy(fwd_hbm.at[eidx], erows, isem)
    cb.wait()
    wb = pltpu.async_copy(brows, out_hbm.at[pl.ds(base, bpw), pl.ds(0, D)], wsem)
    ce.wait()
    we = pltpu.async_copy(erows, out_hbm.at[pl.ds(base, bpw), pl.ds(D, D)], wsem)
    wb.wait()
    we.wait()

  return k


def kernel(begin, end, forward, backward):
  B, S, D = forward.shape
  begin_f = begin.reshape(B).astype(jnp.int32)
  end_f = end.reshape(B).astype(jnp.int32)
  fwd = forward.reshape(B * S, D)
  bwd = backward.reshape(B * S, D)
  return _build(B, S, D)(begin_f, end_f, fwd, bwd)
